# baseline (speedup 1.0000x reference)
import functools

import jax
import jax.numpy as jnp
from jax import lax
from jax.experimental import pallas as pl
from jax.experimental.pallas import tpu as pltpu
from jax.experimental.pallas import tpu_sc as plsc

NC = 2
NS = 16
NW = NC * NS
BPC = 2
NBUF = 2


@functools.lru_cache(maxsize=None)
def _make_gather(b, s, v, d):
    mesh = plsc.VectorSubcoreMesh(core_axis_name="c", subcore_axis_name="s")
    b_per_w = b // NW
    n_chunks = b_per_w // BPC
    chunk = BPC * s
    n_rows = b_per_w * s

    @functools.partial(
        pl.kernel,
        out_type=jax.ShapeDtypeStruct((b, s, d), jnp.float32),
        mesh=mesh,
        scratch_types=[
            pltpu.VMEM((n_rows,), jnp.int32),
            pltpu.VMEM((NBUF, BPC, s, d), jnp.float32),
            pltpu.SemaphoreType.DMA((NBUF,)),
            pltpu.SemaphoreType.DMA((NBUF,)),
        ],
    )
    def gather_kernel(idx_hbm, table_hbm, out_hbm, idx_v, rows_v, gsem, osem):
        wid = lax.axis_index("s") * NC + lax.axis_index("c")
        base = wid * n_rows
        pltpu.sync_copy(idx_hbm.at[pl.ds(base, n_rows)], idx_v)

        def issue(c, bb):
            @pl.loop(0, chunk // 16)
            def _(g):
                iv = idx_v[pl.ds(c * chunk + g * 16, 16)]
                for i in range(16):
                    k = g * 16 + i
                    pltpu.async_copy(
                        table_hbm.at[pl.ds(iv[i], 1)],
                        rows_v.at[bb].at[k // s].at[pl.ds(k % s, 1)],
                        gsem.at[bb],
                    )

        def drain(bb):
            for j in range(BPC):
                pltpu.make_async_copy(
                    table_hbm.at[pl.ds(0, s)],
                    rows_v.at[bb].at[j],
                    gsem.at[bb],
                ).wait()

        outs = [None] * n_chunks
        for c in range(min(NBUF, n_chunks)):
            issue(c, c)
        for c in range(n_chunks):
            bb = c % NBUF
            drain(bb)
            outs[c] = pltpu.async_copy(
                rows_v.at[bb],
                out_hbm.at[pl.ds(wid * b_per_w + c * BPC, BPC)],
                osem.at[bb])
            if c + NBUF < n_chunks:
                outs[c].wait()
                issue(c + NBUF, bb)
        for c in range(max(0, n_chunks - NBUF), n_chunks):
            outs[c].wait()

    return gather_kernel


def kernel(sequence, table):
    b, s = sequence.shape
    v, d = table.shape
    flat = sequence.reshape(-1).astype(jnp.int32)
    assert b % (NW * BPC) == 0 and (BPC * s) % 16 == 0
    out = _make_gather(b, s, v, d)(flat, table)
    return out
